# baseline (speedup 1.0000x reference)
import jax
import jax.numpy as jnp
from jax import lax
from jax.experimental import pallas as pl

_NUM_E = 8192
_HALF = 4096
_DIM = 32
_BETA = 0.25
_BLK = 256
_N_ROWS = 16384
_GRID = _N_ROWS // _BLK
_DN = (((1,), (0,)), ((), ()))


def _argmin_first(d, width):
    m = jnp.min(d, axis=1, keepdims=True)
    iota = lax.broadcasted_iota(jnp.int32, (_BLK, width), 1)
    idx = jnp.min(jnp.where(d == m, iota, width), axis=1, keepdims=True)
    return m, idx


def _vq_block(a16_ref, flat_ref, emb_t_ref, emb_ref, zl2_ref, el2_ref,
              z_ref, zq_ref, counts_ref, sqsum_ref):
    i = pl.program_id(0)
    f = flat_ref[...]
    inner = lax.dot_general(
        a16_ref[...], emb_t_ref[...].astype(jnp.bfloat16), _DN,
        preferred_element_type=jnp.float32)
    dist = (zl2_ref[...] + el2_ref[...]) - inner * 2.0

    m1, i1 = _argmin_first(dist[:, :_HALF], _HALF)
    m2, i2 = _argmin_first(dist[:, _HALF:], _HALF)
    m1r = m1.astype(jnp.bfloat16).astype(jnp.float32)
    pick = jnp.where(m2 < m1r, i2 + _HALF, i1)
    z_ref[0, 0, :] = pick[:, 0]

    iota = lax.broadcasted_iota(jnp.int32, (_BLK, _NUM_E), 1)
    onehot = (iota == pick).astype(jnp.float32)
    zq_rows = lax.dot_general(
        onehot, emb_ref[...], _DN,
        precision=lax.Precision.HIGHEST,
        preferred_element_type=jnp.float32)
    zq_ref[...] = f + (zq_rows - f)

    @pl.when(i == 0)
    def _init():
        counts_ref[...] = jnp.zeros_like(counts_ref)
        sqsum_ref[...] = jnp.zeros_like(sqsum_ref)

    counts_ref[...] += jnp.sum(onehot, axis=0, keepdims=True)
    sqsum_ref[...] += jnp.sum((f - zq_rows) ** 2).reshape(1, 1)


@jax.jit
def kernel(z_e, embedding):
    zp = jnp.transpose(z_e, (0, 2, 3, 1))
    flat = zp.reshape(-1, _DIM)
    a16 = flat.astype(jnp.bfloat16)
    z_l2 = jnp.sum(zp ** 2, axis=3).reshape(-1, 1)
    e_l2 = jnp.sum(embedding ** 2, axis=1).reshape(1, _NUM_E)
    emb_t = embedding.T

    grid_spec = pl.GridSpec(
        grid=(_GRID,),
        in_specs=[
            pl.BlockSpec((_BLK, _DIM), lambda i: (i, 0)),
            pl.BlockSpec((_BLK, _DIM), lambda i: (i, 0)),
            pl.BlockSpec((_DIM, _NUM_E), lambda i: (0, 0)),
            pl.BlockSpec((_NUM_E, _DIM), lambda i: (0, 0)),
            pl.BlockSpec((_BLK, 1), lambda i: (i, 0)),
            pl.BlockSpec((1, _NUM_E), lambda i: (0, 0)),
        ],
        out_specs=[
            pl.BlockSpec((1, 1, _BLK), lambda i: (i, 0, 0)),
            pl.BlockSpec((_BLK, _DIM), lambda i: (i, 0)),
            pl.BlockSpec((1, _NUM_E), lambda i: (0, 0)),
            pl.BlockSpec((1, 1), lambda i: (0, 0)),
        ],
    )
    z3, zq, counts, sqsum = pl.pallas_call(
        _vq_block,
        grid_spec=grid_spec,
        out_shape=[
            jax.ShapeDtypeStruct((_GRID, 1, _BLK), jnp.int32),
            jax.ShapeDtypeStruct((_N_ROWS, _DIM), jnp.float32),
            jax.ShapeDtypeStruct((1, _NUM_E), jnp.float32),
            jax.ShapeDtypeStruct((1, 1), jnp.float32),
        ],
    )(a16, flat, emb_t, embedding, z_l2, e_l2)

    z = z3.reshape(_N_ROWS)
    z_q_out = jnp.transpose(zq.reshape(zp.shape), (0, 3, 1, 2))
    mse = sqsum[0, 0] / (_N_ROWS * _DIM)
    vq_loss = _BETA * mse + mse
    avg_probs = counts.reshape(_NUM_E) / _N_ROWS
    perplexity = jnp.exp(-jnp.sum(avg_probs * jnp.log(avg_probs + 1e-10)))
    return (vq_loss, z_q_out, perplexity, z)


# speedup vs baseline: 1.8375x; 1.8375x over previous
import jax
import jax.numpy as jnp
from jax import lax
from jax.experimental import pallas as pl

_NUM_E = 8192
_HALF = 4096
_DIM = 32
_BETA = 0.25
_BLK = 256
_N_ROWS = 16384
_GRID = _N_ROWS // _BLK
_DN = (((1,), (0,)), ((), ()))


def _argmin_first(d, width):
    m = jnp.min(d, axis=1, keepdims=True)
    iota = lax.broadcasted_iota(jnp.int32, (_BLK, width), 1)
    idx = jnp.min(jnp.where(d == m, iota, width), axis=1, keepdims=True)
    return m, idx


def _vq_block(a16_ref, flat_ref, emb_t_ref, emb_ref, zl2_ref, el2_ref,
              z_ref, zq_ref, counts_ref, sqsum_ref):
    i = pl.program_id(0)
    f = flat_ref[...]
    inner = lax.dot_general(
        a16_ref[...], emb_t_ref[...].astype(jnp.bfloat16), _DN,
        preferred_element_type=jnp.float32)
    dist = (zl2_ref[...] + el2_ref[...]) - inner * 2.0

    m1, i1 = _argmin_first(dist[:, :_HALF], _HALF)
    m2, i2 = _argmin_first(dist[:, _HALF:], _HALF)
    m1r = m1.astype(jnp.bfloat16).astype(jnp.float32)
    pick = jnp.where(m2 < m1r, i2 + _HALF, i1)
    z_ref[0, 0, :] = pick[:, 0]

    iota = lax.broadcasted_iota(jnp.int32, (_BLK, _NUM_E), 1)
    onehot = (iota == pick).astype(jnp.bfloat16)
    emb = emb_ref[...]
    emb_hi = emb.astype(jnp.bfloat16)
    emb_lo = (emb - emb_hi.astype(jnp.float32)).astype(jnp.bfloat16)
    zq_rows = (
        lax.dot_general(onehot, emb_hi, _DN,
                        preferred_element_type=jnp.float32)
        + lax.dot_general(onehot, emb_lo, _DN,
                          preferred_element_type=jnp.float32))
    zq_ref[...] = f + (zq_rows - f)

    @pl.when(i == 0)
    def _init():
        counts_ref[...] = jnp.zeros_like(counts_ref)
        sqsum_ref[...] = jnp.zeros_like(sqsum_ref)

    counts_ref[...] += jnp.sum(onehot.astype(jnp.float32), axis=0,
                               keepdims=True)
    sqsum_ref[...] += jnp.sum((f - zq_rows) ** 2).reshape(1, 1)


@jax.jit
def kernel(z_e, embedding):
    zp = jnp.transpose(z_e, (0, 2, 3, 1))
    flat = zp.reshape(-1, _DIM)
    a16 = flat.astype(jnp.bfloat16)
    z_l2 = jnp.sum(zp ** 2, axis=3).reshape(-1, 1)
    e_l2 = jnp.sum(embedding ** 2, axis=1).reshape(1, _NUM_E)
    emb_t = embedding.T

    grid_spec = pl.GridSpec(
        grid=(_GRID,),
        in_specs=[
            pl.BlockSpec((_BLK, _DIM), lambda i: (i, 0)),
            pl.BlockSpec((_BLK, _DIM), lambda i: (i, 0)),
            pl.BlockSpec((_DIM, _NUM_E), lambda i: (0, 0)),
            pl.BlockSpec((_NUM_E, _DIM), lambda i: (0, 0)),
            pl.BlockSpec((_BLK, 1), lambda i: (i, 0)),
            pl.BlockSpec((1, _NUM_E), lambda i: (0, 0)),
        ],
        out_specs=[
            pl.BlockSpec((1, 1, _BLK), lambda i: (i, 0, 0)),
            pl.BlockSpec((_BLK, _DIM), lambda i: (i, 0)),
            pl.BlockSpec((1, _NUM_E), lambda i: (0, 0)),
            pl.BlockSpec((1, 1), lambda i: (0, 0)),
        ],
    )
    z3, zq, counts, sqsum = pl.pallas_call(
        _vq_block,
        grid_spec=grid_spec,
        out_shape=[
            jax.ShapeDtypeStruct((_GRID, 1, _BLK), jnp.int32),
            jax.ShapeDtypeStruct((_N_ROWS, _DIM), jnp.float32),
            jax.ShapeDtypeStruct((1, _NUM_E), jnp.float32),
            jax.ShapeDtypeStruct((1, 1), jnp.float32),
        ],
    )(a16, flat, emb_t, embedding, z_l2, e_l2)

    z = z3.reshape(_N_ROWS)
    z_q_out = jnp.transpose(zq.reshape(zp.shape), (0, 3, 1, 2))
    mse = sqsum[0, 0] / (_N_ROWS * _DIM)
    vq_loss = _BETA * mse + mse
    avg_probs = counts.reshape(_NUM_E) / _N_ROWS
    perplexity = jnp.exp(-jnp.sum(avg_probs * jnp.log(avg_probs + 1e-10)))
    return (vq_loss, z_q_out, perplexity, z)


# speedup vs baseline: 2.7744x; 1.5099x over previous
import functools

import jax
import jax.numpy as jnp
from jax import lax
from jax.experimental import pallas as pl
from jax.experimental.pallas import tpu as pltpu
from jax.experimental.pallas import tpu_sc as plsc

_NUM_E = 8192
_HALF = 4096
_DIM = 32
_BETA = 0.25
_BLK = 256
_N_ROWS = 16384
_GRID = _N_ROWS // _BLK
_DN = (((1,), (0,)), ((), ()))

_SC_INFO = plsc.get_sparse_core_info()
_NC = _SC_INFO.num_cores
_NS = _SC_INFO.num_subcores
_NW = _NC * _NS
_ROWS_PER_W = _N_ROWS // _NW


def _argmin_first(d, width):
    m = jnp.min(d, axis=1, keepdims=True)
    iota = lax.broadcasted_iota(jnp.int32, (_BLK, width), 1)
    idx = jnp.min(jnp.where(d == m, iota, width), axis=1, keepdims=True)
    return m, idx


def _tc_block(a16x2_ref, emb_t_ref, zl2_ref, el2_ref, z_ref, sqsum_ref):
    i = pl.program_id(0)
    inner2 = lax.dot_general(
        a16x2_ref[...], emb_t_ref[...].astype(jnp.bfloat16), _DN,
        preferred_element_type=jnp.float32)
    dist = (zl2_ref[...] + el2_ref[...]) - inner2

    m1, i1 = _argmin_first(dist[:, :_HALF], _HALF)
    m2, i2 = _argmin_first(dist[:, _HALF:], _HALF)
    m1r = m1.astype(jnp.bfloat16).astype(jnp.float32)
    upper = m2 < m1r
    pick = jnp.where(upper, i2 + _HALF, i1)
    z_ref[0, 0, :] = pick[:, 0]

    @pl.when(i == 0)
    def _init():
        sqsum_ref[...] = jnp.zeros_like(sqsum_ref)

    m_pick = jnp.where(upper, m2, m1)
    sqsum_ref[...] += jnp.sum(m_pick).reshape(1, 1)


def _sc_gather_counts(table_hbm, idx_hbm, ones_hbm, zeros_hbm,
                      zq_hbm, counts_hbm,
                      idx_v, rows_v, ones_v, shared_counts, sem):
    cid = lax.axis_index("c")
    sid = lax.axis_index("s")
    wid = sid * _NC + cid
    base = wid * _ROWS_PER_W

    @pl.when(sid == 0)
    def _zero_counts():
        pltpu.sync_copy(zeros_hbm, shared_counts)

    pltpu.sync_copy(idx_hbm.at[pl.ds(base, _ROWS_PER_W)], idx_v)
    pltpu.sync_copy(ones_hbm.at[pl.ds(base, _ROWS_PER_W)], ones_v)
    pltpu.async_copy(table_hbm.at[idx_v], rows_v, sem).wait()
    pltpu.sync_copy(rows_v, zq_hbm.at[pl.ds(base, _ROWS_PER_W)])

    plsc.subcore_barrier()
    pltpu.sync_copy(ones_v, shared_counts.at[idx_v], add=True)
    plsc.subcore_barrier()

    @pl.when(sid == 0)
    def _publish():
        pltpu.sync_copy(shared_counts, counts_hbm.at[cid])


@jax.jit
def kernel(z_e, embedding):
    zp = jnp.transpose(z_e, (0, 2, 3, 1))
    flat = zp.reshape(-1, _DIM)
    a16x2 = flat.astype(jnp.bfloat16) * jnp.bfloat16(2.0)
    z_l2 = jnp.sum(zp ** 2, axis=3).reshape(-1, 1)
    e_l2 = jnp.sum(embedding ** 2, axis=1).reshape(1, _NUM_E)
    emb_t = embedding.T

    grid_spec = pl.GridSpec(
        grid=(_GRID,),
        in_specs=[
            pl.BlockSpec((_BLK, _DIM), lambda i: (i, 0)),
            pl.BlockSpec((_DIM, _NUM_E), lambda i: (0, 0)),
            pl.BlockSpec((_BLK, 1), lambda i: (i, 0)),
            pl.BlockSpec((1, _NUM_E), lambda i: (0, 0)),
        ],
        out_specs=[
            pl.BlockSpec((1, 1, _BLK), lambda i: (i, 0, 0)),
            pl.BlockSpec((1, 1), lambda i: (0, 0)),
        ],
    )
    z3, sqsum = pl.pallas_call(
        _tc_block,
        grid_spec=grid_spec,
        out_shape=[
            jax.ShapeDtypeStruct((_GRID, 1, _BLK), jnp.int32),
            jax.ShapeDtypeStruct((1, 1), jnp.float32),
        ],
    )(a16x2, emb_t, z_l2, e_l2)

    z = z3.reshape(_N_ROWS)

    sc_kernel = pl.kernel(
        _sc_gather_counts,
        mesh=plsc.VectorSubcoreMesh(core_axis_name="c", subcore_axis_name="s"),
        out_type=(
            jax.ShapeDtypeStruct((_N_ROWS, 128), jnp.float32),
            jax.ShapeDtypeStruct((_NC, _NUM_E), jnp.float32),
        ),
        scratch_types=[
            pltpu.VMEM((_ROWS_PER_W,), jnp.int32),
            pltpu.VMEM((_ROWS_PER_W, 128), jnp.float32),
            pltpu.VMEM((_ROWS_PER_W,), jnp.float32),
            pltpu.VMEM_SHARED((_NUM_E,), jnp.float32),
            pltpu.SemaphoreType.DMA,
        ],
    )
    table_pad = jnp.pad(embedding, ((0, 0), (0, 128 - _DIM)))
    zq_pad, counts2 = sc_kernel(
        table_pad, z,
        jnp.ones((_N_ROWS,), jnp.float32),
        jnp.zeros((_NUM_E,), jnp.float32),
    )

    zq_rows = zq_pad[:, :_DIM]
    z_q_st = flat + (zq_rows - flat)
    z_q_out = jnp.transpose(z_q_st.reshape(zp.shape), (0, 3, 1, 2))
    mse = sqsum[0, 0] / (_N_ROWS * _DIM)
    vq_loss = _BETA * mse + mse
    counts = counts2[0] + counts2[1]
    avg_probs = counts / _N_ROWS
    perplexity = jnp.exp(-jnp.sum(avg_probs * jnp.log(avg_probs + 1e-10)))
    return (vq_loss, z_q_out, perplexity, z)


# speedup vs baseline: 2.8160x; 1.0150x over previous
import functools

import jax
import jax.numpy as jnp
from jax import lax
from jax.experimental import pallas as pl
from jax.experimental.pallas import tpu as pltpu
from jax.experimental.pallas import tpu_sc as plsc

_NUM_E = 8192
_HALF = 4096
_DIM = 32
_BETA = 0.25
_BLK = 512
_N_ROWS = 16384
_GRID = _N_ROWS // _BLK
_DN = (((1,), (0,)), ((), ()))

_SC_INFO = plsc.get_sparse_core_info()
_NC = _SC_INFO.num_cores
_NS = _SC_INFO.num_subcores
_NW = _NC * _NS
_ROWS_PER_W = _N_ROWS // _NW


def _argmin_first(d, width):
    m = jnp.min(d, axis=1, keepdims=True)
    iota = lax.broadcasted_iota(jnp.int32, (_BLK, width), 1)
    idx = jnp.min(jnp.where(d == m, iota, width), axis=1, keepdims=True)
    return m, idx


def _tc_block(a16x2_ref, emb_t_ref, zl2_ref, el2_ref, z_ref, sqsum_ref):
    i = pl.program_id(0)
    inner2 = lax.dot_general(
        a16x2_ref[...], emb_t_ref[...], _DN,
        preferred_element_type=jnp.float32)
    dist = (zl2_ref[...] + el2_ref[...]) - inner2

    m1, i1 = _argmin_first(dist[:, :_HALF], _HALF)
    m2, i2 = _argmin_first(dist[:, _HALF:], _HALF)
    m1r = m1.astype(jnp.bfloat16).astype(jnp.float32)
    upper = m2 < m1r
    pick = jnp.where(upper, i2 + _HALF, i1)
    z_ref[0, 0, :] = pick[:, 0]

    @pl.when(i == 0)
    def _init():
        sqsum_ref[...] = jnp.zeros_like(sqsum_ref)

    m_pick = jnp.where(upper, m2, m1)
    sqsum_ref[...] += jnp.sum(m_pick).reshape(1, 1)


def _sc_gather_counts(table_hbm, idx_hbm, ones_hbm, zeros_hbm,
                      zq_hbm, counts_hbm,
                      idx_v, rows_v, ones_v, shared_counts, sem):
    cid = lax.axis_index("c")
    sid = lax.axis_index("s")
    wid = sid * _NC + cid
    base = wid * _ROWS_PER_W

    @pl.when(sid == 0)
    def _zero_counts():
        pltpu.sync_copy(zeros_hbm, shared_counts)

    pltpu.sync_copy(idx_hbm.at[pl.ds(base, _ROWS_PER_W)], idx_v)
    pltpu.sync_copy(ones_hbm.at[pl.ds(base, _ROWS_PER_W)], ones_v)
    pltpu.async_copy(table_hbm.at[idx_v], rows_v, sem).wait()
    pltpu.sync_copy(rows_v, zq_hbm.at[pl.ds(base, _ROWS_PER_W)])

    plsc.subcore_barrier()
    pltpu.sync_copy(ones_v, shared_counts.at[idx_v], add=True)
    plsc.subcore_barrier()

    @pl.when(sid == 0)
    def _publish():
        pltpu.sync_copy(shared_counts, counts_hbm.at[cid])


@jax.jit
def kernel(z_e, embedding):
    zp = jnp.transpose(z_e, (0, 2, 3, 1))
    flat = zp.reshape(-1, _DIM)
    a16x2 = flat.astype(jnp.bfloat16) * jnp.bfloat16(2.0)
    z_l2 = jnp.sum(zp ** 2, axis=3).reshape(-1, 1)
    e_l2 = jnp.sum(embedding ** 2, axis=1).reshape(1, _NUM_E)
    emb_t = embedding.T.astype(jnp.bfloat16)

    grid_spec = pl.GridSpec(
        grid=(_GRID,),
        in_specs=[
            pl.BlockSpec((_BLK, _DIM), lambda i: (i, 0)),
            pl.BlockSpec((_DIM, _NUM_E), lambda i: (0, 0)),
            pl.BlockSpec((_BLK, 1), lambda i: (i, 0)),
            pl.BlockSpec((1, _NUM_E), lambda i: (0, 0)),
        ],
        out_specs=[
            pl.BlockSpec((1, 1, _BLK), lambda i: (i, 0, 0)),
            pl.BlockSpec((1, 1), lambda i: (0, 0)),
        ],
    )
    z3, sqsum = pl.pallas_call(
        _tc_block,
        grid_spec=grid_spec,
        out_shape=[
            jax.ShapeDtypeStruct((_GRID, 1, _BLK), jnp.int32),
            jax.ShapeDtypeStruct((1, 1), jnp.float32),
        ],
    )(a16x2, emb_t, z_l2, e_l2)

    z = z3.reshape(_N_ROWS)

    sc_kernel = pl.kernel(
        _sc_gather_counts,
        mesh=plsc.VectorSubcoreMesh(core_axis_name="c", subcore_axis_name="s"),
        out_type=(
            jax.ShapeDtypeStruct((_N_ROWS, 128), jnp.float32),
            jax.ShapeDtypeStruct((_NC, _NUM_E), jnp.float32),
        ),
        scratch_types=[
            pltpu.VMEM((_ROWS_PER_W,), jnp.int32),
            pltpu.VMEM((_ROWS_PER_W, 128), jnp.float32),
            pltpu.VMEM((_ROWS_PER_W,), jnp.float32),
            pltpu.VMEM_SHARED((_NUM_E,), jnp.float32),
            pltpu.SemaphoreType.DMA,
        ],
    )
    table_pad = jnp.pad(embedding, ((0, 0), (0, 128 - _DIM)))
    zq_pad, counts2 = sc_kernel(
        table_pad, z,
        jnp.ones((_N_ROWS,), jnp.float32),
        jnp.zeros((_NUM_E,), jnp.float32),
    )

    zq_rows = zq_pad[:, :_DIM]
    z_q_st = flat + (zq_rows - flat)
    z_q_out = jnp.transpose(z_q_st.reshape(zp.shape), (0, 3, 1, 2))
    mse = sqsum[0, 0] / (_N_ROWS * _DIM)
    vq_loss = _BETA * mse + mse
    counts = counts2[0] + counts2[1]
    avg_probs = counts / _N_ROWS
    perplexity = jnp.exp(-jnp.sum(avg_probs * jnp.log(avg_probs + 1e-10)))
    return (vq_loss, z_q_out, perplexity, z)
